# baseline (speedup 1.0000x reference)
import functools

import jax
import jax.numpy as jnp
from jax import lax
from jax.experimental import pallas as pl
from jax.experimental.pallas import tpu as pltpu
from jax.experimental.pallas import tpu_sc as plsc

F32 = jnp.float32
I32 = jnp.int32

N = 10000
E = 320000
D_HID = 256
D_OUT = 128

NC, NS = 2, 16
NW = NC * NS
K = 64
WPW = 162
CWPW = 324
EP = NW * WPW * K
ROWS = EP // K
TROWS = 10032
RPT = TROWS // NS
NWB = 11
WBR = RPT // NWB
DTROWS = 10240
DRPT = DTROWS // NS
NBUF = 6
DBUF = 9

_mesh = plsc.VectorSubcoreMesh(core_axis_name="c", subcore_axis_name="s")


def _deg_body(dst_hbm, deg0_out, deg1_out, deg_sh, dst_v, ones_v, wb_v,
              dsem):
    cid = lax.axis_index("c")
    sid = lax.axis_index("s")
    wid = cid * NS + sid

    for l in range(K // 16):
        ones_v[pl.ds(l * 16, 16)] = jnp.full((16,), 1.0, F32)

    def zr(i, carry):
        wb_v[pl.ds(i * 16, 16)] = jnp.zeros((16,), F32)
        return carry
    lax.fori_loop(0, DRPT // 16, zr, 0)

    pltpu.sync_copy(wb_v, deg_sh.at[pl.ds(sid * DRPT, DRPT)])
    plsc.subcore_barrier()

    pltpu.sync_copy(dst_hbm.at[pl.ds(wid * WPW, WPW)], dst_v)

    def step(g, carry):
        for b in range(DBUF):
            j = g * DBUF + b
            pltpu.async_copy(ones_v, deg_sh.at[dst_v.at[j]], dsem, add=True)
        for b in range(DBUF):
            j = g * DBUF + b
            pltpu.make_async_copy(ones_v, deg_sh.at[dst_v.at[j]],
                                  dsem).wait()
        return carry
    lax.fori_loop(0, WPW // DBUF, step, 0)

    plsc.subcore_barrier()
    pltpu.sync_copy(deg_sh.at[pl.ds(sid * DRPT, DRPT)], wb_v)

    @pl.when(cid == 0)
    def _wb0():
        pltpu.sync_copy(wb_v, deg0_out.at[pl.ds(sid * DRPT, DRPT)])

    @pl.when(cid == 1)
    def _wb1():
        pltpu.sync_copy(wb_v, deg1_out.at[pl.ds(sid * DRPT, DRPT)])


_deg_kernel = pl.kernel(
    _deg_body,
    out_type=[jax.ShapeDtypeStruct((DTROWS,), F32)] * 2,
    mesh=_mesh,
    compiler_params=pltpu.CompilerParams(use_tc_tiling_on_sc=False),
    scratch_types=[
        pltpu.VMEM_SHARED((DTROWS,), F32),
        pltpu.VMEM((WPW, K), I32),
        pltpu.VMEM((K,), F32),
        pltpu.VMEM((DRPT,), F32),
        pltpu.SemaphoreType.DMA,
    ],
)


def _conv_body(split_edges, src_hbm, dst_hbm, y_hbm, out_hbm,
               src_v, dst_v, acc_sh, rbuf, gsem, ssem, isem):
    cid = lax.axis_index("c")
    sid = lax.axis_index("s")
    if split_edges:
        nwin = WPW
        slab = (cid * NS + sid) * WPW
        yv = y_hbm
    else:
        nwin = CWPW
        slab = sid * CWPW
        yv = y_hbm.at[cid]
    ngrp = nwin // NBUF
    ov = out_hbm.at[cid]

    pltpu.async_copy(src_hbm.at[pl.ds(slab, NBUF)], src_v.at[0], isem.at[0])
    pltpu.async_copy(dst_hbm.at[pl.ds(slab, NBUF)], dst_v.at[0], isem.at[0])

    def zrow(i, carry):
        for l in range(8):
            rbuf[0, i, pl.ds(l * 16, 16)] = jnp.zeros((16,), F32)
        return carry
    lax.fori_loop(0, WBR, zrow, 0)

    def zcp(q, carry):
        pltpu.async_copy(rbuf.at[0, pl.ds(0, WBR)],
                         acc_sh.at[pl.ds(sid * RPT + q * WBR, WBR), :],
                         ssem.at[0])
        return carry
    lax.fori_loop(0, NWB, zcp, 0)

    def zdr(q, carry):
        pltpu.make_async_copy(
            rbuf.at[0, pl.ds(0, WBR)],
            acc_sh.at[pl.ds(sid * RPT, WBR), :], ssem.at[0]).wait()
        return carry
    lax.fori_loop(0, NWB, zdr, 0)
    plsc.subcore_barrier()

    pltpu.make_async_copy(src_hbm.at[pl.ds(slab, NBUF)], src_v.at[0],
                          isem.at[0]).wait()
    pltpu.make_async_copy(dst_hbm.at[pl.ds(slab, NBUF)], dst_v.at[0],
                          isem.at[0]).wait()
    pltpu.async_copy(src_hbm.at[pl.ds(slab + NBUF, NBUF)], src_v.at[1],
                     isem.at[1])
    pltpu.async_copy(dst_hbm.at[pl.ds(slab + NBUF, NBUF)], dst_v.at[1],
                     isem.at[1])

    def prime(b, carry):
        pltpu.async_copy(yv.at[src_v.at[0, b]], rbuf.at[b], gsem.at[b])
        return carry
    lax.fori_loop(0, NBUF, prime, 0)

    def step(g, carry):
        gb = lax.rem(g, 2)
        nb = lax.rem(g + 1, 2)

        def ph1(b, carry):
            pltpu.make_async_copy(
                yv.at[src_v.at[gb, b]], rbuf.at[b], gsem.at[b]).wait()
            pltpu.async_copy(
                rbuf.at[b], acc_sh.at[dst_v.at[gb, b]], ssem.at[b],
                add=True)
            return carry
        lax.fori_loop(0, NBUF, ph1, 0)

        @pl.when(g + 1 < ngrp)
        def _wi():
            pltpu.make_async_copy(
                src_hbm.at[pl.ds(slab, NBUF)], src_v.at[nb],
                isem.at[nb]).wait()
            pltpu.make_async_copy(
                dst_hbm.at[pl.ds(slab, NBUF)], dst_v.at[nb],
                isem.at[nb]).wait()

        def ph2(b, carry):
            pltpu.make_async_copy(
                rbuf.at[b], acc_sh.at[dst_v.at[gb, b]], ssem.at[b]).wait()

            @pl.when(g + 1 < ngrp)
            def _issue_next():
                pltpu.async_copy(
                    yv.at[src_v.at[nb, b]], rbuf.at[b], gsem.at[b])
            return carry
        lax.fori_loop(0, NBUF, ph2, 0)

        @pl.when(g + 2 < ngrp)
        def _pf():
            base = slab + (g + 2) * NBUF
            pltpu.async_copy(src_hbm.at[pl.ds(base, NBUF)], src_v.at[gb],
                             isem.at[gb])
            pltpu.async_copy(dst_hbm.at[pl.ds(base, NBUF)], dst_v.at[gb],
                             isem.at[gb])
        return carry
    lax.fori_loop(0, ngrp, step, 0)

    plsc.subcore_barrier()
    def wprime(q, carry):
        pltpu.async_copy(
            acc_sh.at[pl.ds(sid * RPT + q * WBR, WBR), :],
            rbuf.at[lax.rem(q, NBUF), pl.ds(0, WBR)],
            gsem.at[lax.rem(q, NBUF)])
        return carry
    lax.fori_loop(0, NBUF, wprime, 0)

    def wchain(q, carry):
        b = lax.rem(q, NBUF)
        pltpu.make_async_copy(
            acc_sh.at[pl.ds(sid * RPT, WBR), :],
            rbuf.at[b, pl.ds(0, WBR)], gsem.at[b]).wait()
        pltpu.async_copy(
            rbuf.at[b, pl.ds(0, WBR)],
            ov.at[pl.ds(sid * RPT + q * WBR, WBR), :], ssem.at[b])

        @pl.when(q + NBUF < NWB)
        def _next_rd():
            pltpu.make_async_copy(
                rbuf.at[b, pl.ds(0, WBR)],
                ov.at[pl.ds(sid * RPT, WBR), :], ssem.at[b]).wait()
            pltpu.async_copy(
                acc_sh.at[pl.ds(sid * RPT + (q + NBUF) * WBR, WBR), :],
                rbuf.at[b, pl.ds(0, WBR)], gsem.at[b])
        return carry
    lax.fori_loop(0, NWB, wchain, 0)

    def wdrain(q, carry):
        b = lax.rem(q, NBUF)
        pltpu.make_async_copy(
            rbuf.at[b, pl.ds(0, WBR)],
            ov.at[pl.ds(sid * RPT, WBR), :], ssem.at[b]).wait()
        return carry
    lax.fori_loop(NWB - NBUF, NWB, wdrain, 0)


def _make_conv(split_edges, y_shape):
    return pl.kernel(
        functools.partial(_conv_body, split_edges),
        out_type=jax.ShapeDtypeStruct((NC, TROWS, 128), F32),
        mesh=_mesh,
        compiler_params=pltpu.CompilerParams(use_tc_tiling_on_sc=False),
        scratch_types=[
            pltpu.VMEM((2, NBUF, K), I32),
            pltpu.VMEM((2, NBUF, K), I32),
            pltpu.VMEM_SHARED((TROWS, 128), F32),
            pltpu.VMEM((NBUF, K, 128), F32),
            pltpu.SemaphoreType.DMA((NBUF,)),
            pltpu.SemaphoreType.DMA((NBUF,)),
            pltpu.SemaphoreType.DMA((2,)),
        ],
    )


_conv1_kernel = _make_conv(False, (NC, N, 128))
_conv2_kernel = _make_conv(True, (N, 128))


RB = 2000
GRID = N // RB


def _dinv_block(d0_ref, d1_ref):
    return lax.rsqrt(d0_ref[...] + d1_ref[...] + 1.0)


def _tc1_body(x_ref, d0_ref, d1_ref, w1_ref, y1_ref):
    dinv = _dinv_block(d0_ref, d1_ref)
    xw = jnp.dot(x_ref[...], w1_ref[...], preferred_element_type=F32)
    y = xw * dinv
    y1_ref[0] = y[:, :128]
    y1_ref[1] = y[:, 128:]


def _tc2_body(s1_ref, y1_ref, d0_ref, d1_ref, b1_ref, w2_ref, y2_ref):
    dinv = _dinv_block(d0_ref, d1_ref)
    b1 = b1_ref[...]
    h0 = jnp.maximum(dinv * (s1_ref[0] + y1_ref[0]) + b1[:, :128], 0.0)
    h1 = jnp.maximum(dinv * (s1_ref[1] + y1_ref[1]) + b1[:, 128:], 0.0)
    h = jnp.concatenate([h0, h1], axis=1)
    y2_ref[...] = jnp.dot(h, w2_ref[...], preferred_element_type=F32) * dinv


def _tc3_body(s2_ref, y2_ref, d0_ref, d1_ref, b2_ref,
              f1w_ref, f1b_ref, f2w_ref, f2b_ref, o_ref):
    dinv = _dinv_block(d0_ref, d1_ref)
    z = jnp.maximum(dinv * (s2_ref[0] + s2_ref[1] + y2_ref[...])
                    + b2_ref[...], 0.0)
    t = jnp.dot(z, f1w_ref[...], preferred_element_type=F32) + f1b_ref[...]
    p = jnp.where(t > 0.0, t, jnp.exp(t) - 1.0)
    o_ref[...] = jnp.dot(p, f2w_ref[...], preferred_element_type=F32) \
        + f2b_ref[...]


def _row_spec(shape):
    nd = len(shape)
    if nd == 2:
        return pl.BlockSpec((RB, shape[1]), lambda i: (i, 0))
    return pl.BlockSpec((shape[0], RB, shape[2]), lambda i: (0, i, 0))


def _full_spec(shape):
    return pl.BlockSpec(shape, lambda i: (0,) * len(shape))


def _tc_call(body, ins_row, ins_full, out_shapes):
    in_specs = [_row_spec(a.shape) for a in ins_row] + \
               [_full_spec(a.shape) for a in ins_full]
    out_specs = [_row_spec(s.shape) for s in out_shapes]
    outs = pl.pallas_call(
        body,
        grid=(GRID,),
        in_specs=in_specs,
        out_specs=out_specs if len(out_shapes) > 1 else out_specs[0],
        out_shape=out_shapes if len(out_shapes) > 1 else out_shapes[0],
    )(*ins_row, *ins_full)
    return outs


def kernel(x, edge_index, W1, b1, W2, b2, fc1_W, fc1_b, fc2_W, fc2_b):
    ei = edge_index.astype(I32)
    src, dst = ei[0], ei[1]
    pad = EP - E
    fk = jnp.arange(pad, dtype=I32)
    fake_src = (fk * 7) & 8191
    fake_dst = N + (fk & 31)
    srcp = jnp.concatenate([src, fake_src]).reshape(ROWS, K)
    dstp = jnp.concatenate([dst, fake_dst]).reshape(ROWS, K)

    d0, d1 = _deg_kernel(dstp)
    d0 = d0.reshape(DTROWS, 1)
    d1 = d1.reshape(DTROWS, 1)

    sds = jax.ShapeDtypeStruct
    y1 = _tc_call(
        _tc1_body, [x, d0, d1], [W1],
        [sds((NC, N, 128), F32)])

    s1 = _conv1_kernel(srcp, dstp, y1)

    y2 = _tc_call(
        _tc2_body, [s1, y1, d0, d1],
        [b1.reshape(1, D_HID), W2],
        [sds((N, 128), F32)])

    s2 = _conv2_kernel(srcp, dstp, y2)

    out = _tc_call(
        _tc3_body, [s2, y2, d0, d1],
        [b2.reshape(1, 128), fc1_W, fc1_b.reshape(1, 128),
         fc2_W, fc2_b.reshape(1, 128)],
        [sds((N, D_OUT), F32)])
    return out
